# baseline (speedup 1.0000x reference)
import functools

import jax
import jax.numpy as jnp
from jax import lax
from jax.experimental import pallas as pl
from jax.experimental.pallas import tpu as pltpu
from jax.experimental.pallas import tpu_sc as plsc

_FIELD_NUM = 26
_EMBED_DIM = 64
_EMBED_SIZE = _FIELD_NUM * _EMBED_DIM
_BATCH = 16384
_MASK_PAD = 512
_NC, _NS = 2, 16
_NW = _NC * _NS
_ROWS_PER_TILE = _BATCH // _NW
_R = 32
_CHUNKS = _ROWS_PER_TILE // _R
_GROUPS = _R * _MASK_PAD // 16
_UNROLL = 8


def _sc_body(x_hbm, idx_hbm, out_hbm, idx_v, buf_v, sem):
    wid = lax.axis_index("s") * _NC + lax.axis_index("c")
    base = wid * (_ROWS_PER_TILE * _EMBED_SIZE)
    pltpu.sync_copy(idx_hbm, idx_v)
    zeros = jnp.zeros((16,), jnp.float32)

    def chunk_body(c, carry):
        off = base + c * (_R * _EMBED_SIZE)
        pltpu.sync_copy(x_hbm.at[pl.ds(off, _R * _EMBED_SIZE)], buf_v)

        def grp_body(g, carry2):
            for u in range(_UNROLL):
                grp = idx_v[pl.ds((g * _UNROLL + u) * 16, 16)]
                plsc.store_scatter(buf_v, [grp], zeros)
            return carry2

        lax.fori_loop(0, _GROUPS // _UNROLL, grp_body, 0, unroll=False)
        pltpu.sync_copy(buf_v, out_hbm.at[pl.ds(off, _R * _EMBED_SIZE)])
        return carry

    lax.fori_loop(0, _CHUNKS, chunk_body, 0, unroll=False)


def kernel(embed, embed_ele_indices):
    B = embed.shape[0]
    x1d = embed.reshape(-1)
    col = embed_ele_indices.astype(jnp.int32)
    col_pad = jnp.concatenate(
        [col, jnp.broadcast_to(col[0:1], (_MASK_PAD - col.shape[0],))])
    chunk_idx = (
        jnp.arange(_R, dtype=jnp.int32)[:, None] * _EMBED_SIZE
        + col_pad[None, :]
    ).reshape(-1)

    mesh = plsc.VectorSubcoreMesh(core_axis_name="c", subcore_axis_name="s")
    run = functools.partial(
        pl.kernel,
        mesh=mesh,
        out_type=jax.ShapeDtypeStruct((B * _EMBED_SIZE,), jnp.float32),
        scratch_types=[
            pltpu.VMEM((_R * _MASK_PAD,), jnp.int32),
            pltpu.VMEM((_R * _EMBED_SIZE,), jnp.float32),
            pltpu.SemaphoreType.DMA,
        ],
        compiler_params=pltpu.CompilerParams(needs_layout_passes=False),
    )(_sc_body)
    out = run(x1d, chunk_idx)
    return out.reshape(B, _FIELD_NUM, _EMBED_DIM)


# speedup vs baseline: 10.4963x; 10.4963x over previous
import jax
import jax.numpy as jnp
from jax.experimental import pallas as pl

_FIELD_NUM = 26
_EMBED_DIM = 64
_EMBED_SIZE = _FIELD_NUM * _EMBED_DIM
_IDX_PAD = 512
_ROW_BLK = 104


def _row_mask_body(idx_ref, x_ref, o_ref):
    i = pl.program_id(0)
    ids = idx_ref[...]
    pos = jax.lax.broadcasted_iota(jnp.int32, (_ROW_BLK, _IDX_PAD), 0) + i * _ROW_BLK
    hit = jnp.any(pos == ids, axis=1, keepdims=True)
    o_ref[...] = jnp.where(hit, 0.0, x_ref[...])


def kernel(embed, embed_ele_indices):
    B = embed.shape[0]
    x_t = embed.transpose(1, 2, 0).reshape(_EMBED_SIZE, B)
    idx = embed_ele_indices.astype(jnp.int32)
    pad = jnp.full((_IDX_PAD - idx.shape[0],), 2**30, dtype=jnp.int32)
    idx2 = jnp.concatenate([idx, pad]).reshape(1, _IDX_PAD)

    out_t = pl.pallas_call(
        _row_mask_body,
        grid=(_EMBED_SIZE // _ROW_BLK,),
        in_specs=[
            pl.BlockSpec((1, _IDX_PAD), lambda i: (0, 0)),
            pl.BlockSpec((_ROW_BLK, B), lambda i: (i, 0)),
        ],
        out_specs=pl.BlockSpec((_ROW_BLK, B), lambda i: (i, 0)),
        out_shape=jax.ShapeDtypeStruct((_EMBED_SIZE, B), jnp.float32),
    )(idx2, x_t)
    return out_t.reshape(_FIELD_NUM, _EMBED_DIM, B).transpose(2, 0, 1)


# speedup vs baseline: 10.5984x; 1.0097x over previous
import jax
import jax.numpy as jnp
from jax.experimental import pallas as pl

_FIELD_NUM = 26
_EMBED_DIM = 64
_EMBED_SIZE = _FIELD_NUM * _EMBED_DIM
_IDX_PAD = 512
_ROW_BLK = 208


def _row_mask_body(idx_ref, x_ref, o_ref):
    i = pl.program_id(0)
    ids = idx_ref[...]
    pos = jax.lax.broadcasted_iota(jnp.int32, (_ROW_BLK, _IDX_PAD), 0) + i * _ROW_BLK
    hit = jnp.any(pos == ids, axis=1, keepdims=True)
    o_ref[...] = jnp.where(hit, 0.0, x_ref[...])


def kernel(embed, embed_ele_indices):
    B = embed.shape[0]
    x_t = embed.transpose(1, 2, 0).reshape(_EMBED_SIZE, B)
    idx = embed_ele_indices.astype(jnp.int32)
    pad = jnp.full((_IDX_PAD - idx.shape[0],), 2**30, dtype=jnp.int32)
    idx2 = jnp.concatenate([idx, pad]).reshape(1, _IDX_PAD)

    out_t = pl.pallas_call(
        _row_mask_body,
        grid=(_EMBED_SIZE // _ROW_BLK,),
        in_specs=[
            pl.BlockSpec((1, _IDX_PAD), lambda i: (0, 0)),
            pl.BlockSpec((_ROW_BLK, B), lambda i: (i, 0)),
        ],
        out_specs=pl.BlockSpec((_ROW_BLK, B), lambda i: (i, 0)),
        out_shape=jax.ShapeDtypeStruct((_EMBED_SIZE, B), jnp.float32),
    )(idx2, x_t)
    return out_t.reshape(_FIELD_NUM, _EMBED_DIM, B).transpose(2, 0, 1)
